# Initial kernel scaffold; baseline (speedup 1.0000x reference)
import jax
import jax.numpy as jnp
from jax.experimental import pallas as pl


def kernel(x, table):
    raise NotImplementedError("write your pallas kernel here")


# speedup vs baseline: 1.0010x; 1.0010x over previous
import math

import jax
import jax.numpy as jnp
from jax import lax
from jax.experimental import pallas as pl
from jax.experimental.pallas import tpu as pltpu
from jax.experimental.pallas import tpu_sc as plsc

VOCAB = 100000
DIM = 1024
B = 4
S = 2048
N = B * S

NC = 2
NS = 16
LANES = 16
NW = NC * NS
PER_W = N // NW
CHUNK = 32
NCHUNK = PER_W // CHUNK
SCALE = float(math.sqrt(DIM))


def _sc_body(idx_hbm, table_hbm, out_hbm, idx_v, rows_v, sem):
    wid = lax.axis_index("s") * NC + lax.axis_index("c")
    base = wid * PER_W
    pltpu.sync_copy(idx_hbm.at[pl.ds(base, PER_W)], idx_v)

    for c in range(NCHUNK):
        pltpu.async_copy(
            table_hbm.at[idx_v.at[pl.ds(c * CHUNK, CHUNK)]], rows_v, sem
        ).wait()

        def scale_row(r, _):
            for k in range(DIM // LANES):
                sl = pl.ds(k * LANES, LANES)
                rows_v[r, sl] = rows_v[r, sl] * SCALE
            return 0

        lax.fori_loop(0, CHUNK, scale_row, 0)

        pltpu.sync_copy(rows_v, out_hbm.at[pl.ds(base + c * CHUNK, CHUNK)])


def _gather_scaled(idx_flat, table):
    mesh = plsc.VectorSubcoreMesh(
        core_axis_name="c", subcore_axis_name="s", num_cores=NC, num_subcores=NS
    )
    return pl.kernel(
        _sc_body,
        out_type=jax.ShapeDtypeStruct((N, DIM), jnp.float32),
        mesh=mesh,
        scratch_types=[
            pltpu.VMEM((PER_W,), jnp.int32),
            pltpu.VMEM((CHUNK, DIM), jnp.float32),
            pltpu.SemaphoreType.DMA,
        ],
    )(idx_flat, table)


def kernel(x, table):
    out = _gather_scaled(x.reshape(N), table)
    return out.reshape(B, S, DIM)


# speedup vs baseline: 1.3322x; 1.3309x over previous
import math

import jax
import jax.numpy as jnp
from jax import lax
from jax.experimental import pallas as pl
from jax.experimental.pallas import tpu as pltpu
from jax.experimental.pallas import tpu_sc as plsc

VOCAB = 100000
DIM = 1024
B = 4
S = 2048
N = B * S

NC = 2
NS = 16
LANES = 16
NW = NC * NS
PER_W = N // NW
CHUNK = 16
NCHUNK = PER_W // CHUNK
NBUF = 2
SCALE = float(math.sqrt(DIM))


def _sc_body(idx_hbm, table_hbm, out_hbm,
             idx_v, g0, g1, s0, s1, gsem0, gsem1, ssem0, ssem1):
    gbufs = (g0, g1)
    sbufs = (s0, s1)
    gsems = (gsem0, gsem1)
    ssems = (ssem0, ssem1)
    wid = lax.axis_index("s") * NC + lax.axis_index("c")
    base = wid * PER_W
    pltpu.sync_copy(idx_hbm.at[pl.ds(base, PER_W)], idx_v)

    def gather(j, b):
        h = pltpu.make_async_copy(
            table_hbm.at[idx_v.at[pl.ds(j * CHUNK, CHUNK)]], gbufs[b], gsems[b]
        )
        h.start()
        return h

    pending_g = [gather(j, j) for j in range(NBUF)]
    pending_s = [None] * NBUF

    for i in range(NCHUNK):
        b = i % NBUF
        pending_g[b].wait()
        if pending_s[b] is not None:
            pending_s[b].wait()

        def scale_row(r, _):
            for k in range(DIM // LANES):
                sl = pl.ds(k * LANES, LANES)
                sbufs[b][r, sl] = gbufs[b][r, sl] * SCALE
            return 0

        lax.fori_loop(0, CHUNK, scale_row, 0)

        hs = pltpu.make_async_copy(
            sbufs[b], out_hbm.at[pl.ds(base + i * CHUNK, CHUNK)], ssems[b]
        )
        hs.start()
        pending_s[b] = hs

        j = i + NBUF
        if j < NCHUNK:
            pending_g[b] = gather(j, b)

    for h in pending_s:
        h.wait()


def _gather_scaled(idx_flat, table):
    mesh = plsc.VectorSubcoreMesh(
        core_axis_name="c", subcore_axis_name="s", num_cores=NC, num_subcores=NS
    )
    return pl.kernel(
        _sc_body,
        out_type=jax.ShapeDtypeStruct((N, DIM), jnp.float32),
        mesh=mesh,
        scratch_types=[
            pltpu.VMEM((PER_W,), jnp.int32),
            pltpu.VMEM((CHUNK, DIM), jnp.float32),
            pltpu.VMEM((CHUNK, DIM), jnp.float32),
            pltpu.VMEM((CHUNK, DIM), jnp.float32),
            pltpu.VMEM((CHUNK, DIM), jnp.float32),
            pltpu.SemaphoreType.DMA,
            pltpu.SemaphoreType.DMA,
            pltpu.SemaphoreType.DMA,
            pltpu.SemaphoreType.DMA,
        ],
    )(idx_flat, table)


def kernel(x, table):
    out = _gather_scaled(x.reshape(N), table)
    return out.reshape(B, S, DIM)


# speedup vs baseline: 1.3480x; 1.0119x over previous
import math

import jax
import jax.numpy as jnp
from jax import lax
from jax.experimental import pallas as pl
from jax.experimental.pallas import tpu as pltpu
from jax.experimental.pallas import tpu_sc as plsc

VOCAB = 100000
DIM = 1024
B = 4
S = 2048
N = B * S

NC = 2
NS = 16
LANES = 16
NW = NC * NS
PER_W = N // NW
CHUNK = 16
NCHUNK = PER_W // CHUNK
NGBUF = 4
NSBUF = 3
SCALE = float(math.sqrt(DIM))


def _sc_body(idx_hbm, table_hbm, out_hbm, idx_v, *rest):
    gbufs = rest[:NGBUF]
    sbufs = rest[NGBUF:NGBUF + NSBUF]
    gsems = rest[NGBUF + NSBUF:2 * NGBUF + NSBUF]
    ssems = rest[2 * NGBUF + NSBUF:]
    wid = lax.axis_index("s") * NC + lax.axis_index("c")
    base = wid * PER_W
    pltpu.sync_copy(idx_hbm.at[pl.ds(base, PER_W)], idx_v)

    def gather(j, b):
        h = pltpu.make_async_copy(
            table_hbm.at[idx_v.at[pl.ds(j * CHUNK, CHUNK)]], gbufs[b], gsems[b]
        )
        h.start()
        return h

    pending_g = [gather(j, j) for j in range(NGBUF)]
    pending_s = [None] * NSBUF

    for i in range(NCHUNK):
        g = i % NGBUF
        s = i % NSBUF
        pending_g[g].wait()
        if pending_s[s] is not None:
            pending_s[s].wait()

        def scale_row(r, _):
            for k in range(DIM // LANES):
                sl = pl.ds(k * LANES, LANES)
                sbufs[s][r, sl] = gbufs[g][r, sl] * SCALE
            return 0

        lax.fori_loop(0, CHUNK, scale_row, 0)

        hs = pltpu.make_async_copy(
            sbufs[s], out_hbm.at[pl.ds(base + i * CHUNK, CHUNK)], ssems[s]
        )
        hs.start()
        pending_s[s] = hs

        j = i + NGBUF
        if j < NCHUNK:
            pending_g[g] = gather(j, g)

    for h in pending_s:
        h.wait()


def _gather_scaled(idx_flat, table):
    mesh = plsc.VectorSubcoreMesh(
        core_axis_name="c", subcore_axis_name="s", num_cores=NC, num_subcores=NS
    )
    return pl.kernel(
        _sc_body,
        out_type=jax.ShapeDtypeStruct((N, DIM), jnp.float32),
        mesh=mesh,
        scratch_types=(
            [pltpu.VMEM((PER_W,), jnp.int32)]
            + [pltpu.VMEM((CHUNK, DIM), jnp.float32)] * (NGBUF + NSBUF)
            + [pltpu.SemaphoreType.DMA] * (NGBUF + NSBUF)
        ),
    )(idx_flat, table)


def kernel(x, table):
    out = _gather_scaled(x.reshape(N), table)
    return out.reshape(B, S, DIM)
